# Initial kernel scaffold; baseline (speedup 1.0000x reference)
import jax
import jax.numpy as jnp
from jax.experimental import pallas as pl


def kernel(input_labels, pos_labels, neg_labels, in_embed):
    raise NotImplementedError("write your pallas kernel here")


# speedup vs baseline: 11.4055x; 11.4055x over previous
import functools

import jax
import jax.numpy as jnp
from jax import lax
from jax.experimental import pallas as pl
from jax.experimental.pallas import tpu as pltpu
from jax.experimental.pallas import tpu_sc as plsc

NC = 2
NS = 16
NW = NC * NS
L = 16


def kernel(input_labels, pos_labels, neg_labels, in_embed):
    B = input_labels.shape[0]
    P = pos_labels.shape[1]
    N = neg_labels.shape[1]
    V, D = in_embed.shape

    tot = 1 + P + N
    KPAD = ((tot + 31) // 32) * 32
    NGROUP = KPAD // L
    CB = 4
    CHUNK_ROWS = CB * KPAD
    NGATHER = CHUNK_ROWS // 128
    b_per_w = B // NW
    n_chunks = b_per_w // CB

    idx_all = jnp.concatenate(
        [
            input_labels[:, None].astype(jnp.int32),
            pos_labels.astype(jnp.int32),
            neg_labels.astype(jnp.int32),
            jnp.zeros((B, KPAD - tot), jnp.int32),
        ],
        axis=1,
    )
    idx_hbm = idx_all.reshape(B * KPAD // 128, 128)

    mesh = plsc.VectorSubcoreMesh(core_axis_name="c", subcore_axis_name="s")
    out_types = (
        jax.ShapeDtypeStruct((B,), jnp.float32),
        jax.ShapeDtypeStruct((B,), jnp.float32),
        jax.ShapeDtypeStruct((B,), jnp.float32),
    )
    IDXBLK = 8
    scratch = [
        pltpu.VMEM((2, IDXBLK * NGATHER, 128), jnp.int32),
        pltpu.VMEM((2 * CHUNK_ROWS, D), jnp.float32),
        pltpu.VMEM((b_per_w,), jnp.float32),
        pltpu.VMEM((b_per_w,), jnp.float32),
        pltpu.VMEM((b_per_w,), jnp.float32),
        pltpu.SemaphoreType.DMA,
        pltpu.SemaphoreType.DMA,
    ]

    @functools.partial(
        pl.kernel,
        out_type=out_types,
        mesh=mesh,
        scratch_types=scratch,
        compiler_params=pltpu.CompilerParams(
            needs_layout_passes=False, use_tc_tiling_on_sc=False
        ),
    )
    def sc_k(table, idxr, out_t, out_sp, out_sn,
             idx_v, rows_v, t_v, sp_v, sn_v, sem0, sem1):
        wid = lax.axis_index("s") * NC + lax.axis_index("c")
        sems = (sem0, sem1)
        lanes = lax.iota(jnp.int32, L)
        lane0 = lanes == 0

        def load_idx_block(blk):
            row = wid * (n_chunks * NGATHER) + blk * (IDXBLK * NGATHER)
            pltpu.sync_copy(
                idxr.at[pl.ds(row, IDXBLK * NGATHER)],
                idx_v.at[blk % 2],
            )

        def start_gathers(c, s):
            blk_s = (c // IDXBLK) % 2
            r0 = (c % IDXBLK) * NGATHER
            for j in range(NGATHER):
                pltpu.async_copy(
                    table.at[idx_v.at[blk_s, r0 + j]],
                    rows_v.at[pl.ds(s * CHUNK_ROWS + j * 128, 128)],
                    sems[s],
                )

        def drain_gathers(s):
            for j in range(NGATHER):
                pltpu.make_async_copy(
                    table.at[idx_v.at[0, j]],
                    rows_v.at[pl.ds(s * CHUNK_ROWS + j * 128, 128)],
                    sems[s],
                ).wait()

        NEG_INF = jnp.float32(-1e30)
        zero = jnp.zeros((L,), jnp.float32)
        pmasks = []
        nmasks = []
        for g in range(NGROUP):
            k = lanes + g * L
            pmasks.append((k >= 1) & (k <= P))
            nmasks.append((k >= 1 + P) & (k <= P + N))

        def compute_chunk(c, s):
            for bl in range(CB):
                base = s * CHUNK_ROWS + bl * KPAD
                rowidx = [
                    jnp.full((L,), base + g * L, jnp.int32) + lanes
                    for g in range(NGROUP)
                ]
                inp_row = jnp.full((L,), base, jnp.int32)

                def body(d, accs):
                    dvec = jnp.full((L,), d, jnp.int32)
                    ev = plsc.load_gather(rows_v, [inp_row, dvec])
                    return tuple(
                        acc + plsc.load_gather(rows_v, [ri, dvec]) * ev
                        for acc, ri in zip(accs, rowidx)
                    )

                accs = lax.fori_loop(
                    0, D, body, tuple(zero for _ in range(NGROUP))
                )

                mp = jnp.full((L,), NEG_INF)
                mn = jnp.full((L,), NEG_INF)
                for g in range(NGROUP):
                    v = accs[g]
                    mp = jnp.maximum(mp, jnp.where(pmasks[g], v, NEG_INF))
                    mn = jnp.maximum(mn, jnp.where(nmasks[g], -v, NEG_INF))
                mps = jnp.max(mp)
                mns = jnp.max(mn)

                spv = zero
                epv = zero
                snv = zero
                env = zero
                for g in range(NGROUP):
                    v = accs[g]
                    spv = spv + jnp.where(pmasks[g], v, 0.0)
                    epv = epv + jnp.where(pmasks[g], jnp.exp(v - mps), 0.0)
                    snv = snv + jnp.where(nmasks[g], -v, 0.0)
                    env = env + jnp.where(nmasks[g], jnp.exp(-v - mns), 0.0)

                i = c * CB + bl
                iv = jnp.full((L,), i, jnp.int32)
                tval = jnp.sum(spv) - P * mps + jnp.sum(snv) - N * mns
                plsc.store_scatter(t_v, [iv], jnp.full((L,), tval), mask=lane0)
                plsc.store_scatter(sp_v, [iv], jnp.full((L,), jnp.sum(epv)), mask=lane0)
                plsc.store_scatter(sn_v, [iv], jnp.full((L,), jnp.sum(env)), mask=lane0)

        load_idx_block(jnp.int32(0))
        for s in range(2):
            start_gathers(jnp.int32(s), s)

        def outer(i, carry):
            for s in range(2):
                c = 2 * i + s
                drain_gathers(s)
                compute_chunk(c, s)

                @pl.when(c + 2 < n_chunks)
                def _():
                    @pl.when((c + 2) % IDXBLK == 0)
                    def _():
                        load_idx_block((c + 2) // IDXBLK)

                    start_gathers(c + 2, s)

            return carry

        lax.fori_loop(0, n_chunks // 2, outer, jnp.int32(0))

        base = wid * b_per_w
        pltpu.sync_copy(t_v, out_t.at[pl.ds(base, b_per_w)])
        pltpu.sync_copy(sp_v, out_sp.at[pl.ds(base, b_per_w)])
        pltpu.sync_copy(sn_v, out_sn.at[pl.ds(base, b_per_w)])

    t, sp, sn = sc_k(in_embed, idx_hbm)

    Pf = float(P)
    Nf = float(N)

    def tc_body(t_ref, sp_ref, sn_ref, o_ref):
        o_ref[...] = (
            -t_ref[...] + Pf * jnp.log(sp_ref[...]) + Nf * jnp.log(sn_ref[...])
        )

    R = 128
    C = B // R
    loss2d = pl.pallas_call(
        tc_body, out_shape=jax.ShapeDtypeStruct((R, C), jnp.float32)
    )(t.reshape(R, C), sp.reshape(R, C), sn.reshape(R, C))
    return loss2d.reshape(B)
